# baseline (speedup 1.0000x reference)
import functools

import numpy as np
import jax
import jax.numpy as jnp
from jax import lax
from jax.experimental import pallas as pl
from jax.experimental.pallas import tpu as pltpu
from jax.experimental.pallas import tpu_sc as plsc

_B = 16384
_F = 26
_D = 16
_EMB = _F * _D
_NIX = _F * (_F - 1) // 2
_FEAT_VOCAB = 100000
_PACK = 128 // _D

_K_OF = np.zeros((_F, _F), dtype=np.int64)
_k = 0
for _i in range(_F - 1):
    for _j in range(_i + 1, _F):
        _K_OF[_i, _j] = _k
        _k += 1
_PERM = np.array([_K_OF[f, f + o] for o in range(1, _F) for f in range(_F - o)])


@functools.cache
def _make_sc_gather():
    info = plsc.get_sparse_core_info()
    nw = info.num_cores * info.num_subcores
    bpw = _B // nw
    cs = bpw // 2

    mesh = plsc.VectorSubcoreMesh(core_axis_name="c", subcore_axis_name="s")

    @functools.partial(
        pl.kernel,
        out_type=jax.ShapeDtypeStruct((_F, _B, _D), jnp.float32),
        mesh=mesh,
        scratch_types=[
            pltpu.VMEM((cs,), jnp.int32),
            pltpu.VMEM((cs,), jnp.int32),
            pltpu.VMEM((cs,), jnp.int32),
            pltpu.VMEM((cs, 128), jnp.float32),
            pltpu.VMEM((cs, _D), jnp.float32),
            pltpu.SemaphoreType.DMA,
        ],
    )
    def gather_k(idx_hbm, uid_hbm, ftab_hbm, out_hbm, idx_v, row_v, mod_v,
                 rows8_v, blk_v, sem):
        wid = lax.axis_index("s") * info.num_cores + lax.axis_index("c")

        def extract():
            def group_body(g, carry):
                mods = mod_v[pl.ds(g * 16, 16)]
                for l in range(16):
                    m = mods[l]
                    s = g * 16 + l
                    blk_v[s, :] = rows8_v[s, pl.ds(m, _D)]
                return carry
            lax.fori_loop(0, cs // 16, group_body, 0)

        def chunk_body(i, carry):
            f = i // 2
            base = wid * bpw + (i % 2) * cs
            pltpu.sync_copy(idx_hbm.at[f, 0, pl.ds(base, cs)], idx_v)
            packed_off = jnp.where(f > 0, (f - 1) * (_FEAT_VOCAB // _PACK), 0)
            for i16 in range(cs // 16):
                sl = idx_v[pl.ds(i16 * 16, 16)]
                row_v[pl.ds(i16 * 16, 16)] = (sl >> 3) + packed_off
                mod_v[pl.ds(i16 * 16, 16)] = (sl & 7) * _D

            @pl.when(f == 0)
            def _():
                pltpu.async_copy(uid_hbm.at[row_v], rows8_v, sem).wait()

            @pl.when(f > 0)
            def _():
                pltpu.async_copy(ftab_hbm.at[row_v], rows8_v, sem).wait()

            extract()
            pltpu.sync_copy(blk_v, out_hbm.at[f, pl.ds(base, cs), :])
            return carry

        lax.fori_loop(0, 2 * _F, chunk_body, 0)

    return gather_k


_TB = 512


def _tc_body(emb_ref, w0a_ref, w0b_ref, b0_ref, w1_ref, b1_ref, w2_ref,
             b2_ref, wo_ref, bo_ref, out_ref):
    v = emb_ref[...]
    et = jnp.transpose(v, (0, 2, 1)).reshape(_EMB, _TB)

    slabs = []
    for o in range(1, _F):
        prod = et[: _EMB - _D * o, :] * et[_D * o :, :]
        slabs.append(jnp.sum(prod.reshape(_F - o, _D, _TB), axis=1))
    cross = jnp.concatenate(slabs, axis=0)

    h = jnp.dot(w0a_ref[...], et, preferred_element_type=jnp.float32)
    h = h + jnp.dot(w0b_ref[...], cross, preferred_element_type=jnp.float32)
    h = jnp.maximum(h + b0_ref[...], 0.0)
    h = jnp.maximum(jnp.dot(w1_ref[...], h, preferred_element_type=jnp.float32)
                    + b1_ref[...], 0.0)
    h = jnp.maximum(jnp.dot(w2_ref[...], h, preferred_element_type=jnp.float32)
                    + b2_ref[...], 0.0)
    o_ = jnp.dot(wo_ref[...], h, preferred_element_type=jnp.float32) + bo_ref[...]
    out_ref[...] = jax.nn.sigmoid(o_)[None]


def _tc_call(emb_t, w0a_t, w0b_t, b0c, w1t, b1c, w2t, b2c, wot, boc):
    nt = _B // _TB
    full = lambda shape: pl.BlockSpec(shape, lambda i: (0, 0))
    return pl.pallas_call(
        _tc_body,
        grid=(nt,),
        in_specs=[
            pl.BlockSpec((_F, _TB, _D), lambda i: (0, i, 0)),
            full((400, _EMB)),
            full((400, _NIX)),
            full((400, 1)),
            full((400, 400)),
            full((400, 1)),
            full((400, 400)),
            full((400, 1)),
            full((1, 400)),
            full((1, 1)),
        ],
        out_specs=pl.BlockSpec((1, 1, _TB), lambda i: (i, 0, 0)),
        out_shape=jax.ShapeDtypeStruct((nt, 1, _TB), jnp.float32),
    )(emb_t, w0a_t, w0b_t, b0c, w1t, b1c, w2t, b2c, wot, boc)


def kernel(user_id, feat_0, feat_1, feat_2, feat_3, feat_4, feat_5, feat_6,
           feat_7, feat_8, feat_9, feat_10, feat_11, feat_12, feat_13,
           feat_14, feat_15, feat_16, feat_17, feat_18, feat_19, feat_20,
           feat_21, feat_22, feat_23, feat_24, uid_table, feat_tables,
           W0, b0, W1, b1, W2, b2, W_out, b_out):
    feats = [feat_0, feat_1, feat_2, feat_3, feat_4, feat_5, feat_6, feat_7,
             feat_8, feat_9, feat_10, feat_11, feat_12, feat_13, feat_14,
             feat_15, feat_16, feat_17, feat_18, feat_19, feat_20, feat_21,
             feat_22, feat_23, feat_24]
    idx_all = jnp.stack([user_id] + feats, axis=0)[:, None, :]
    uid8 = uid_table.reshape(-1, 128)
    ftab8 = feat_tables.reshape(-1, 128)

    emb_t = _make_sc_gather()(idx_all, uid8, ftab8)

    w0a_t = W0[:_EMB].T
    w0b_t = W0[_EMB:][_PERM].T
    out2 = _tc_call(emb_t, w0a_t, w0b_t, b0[:, None], W1.T, b1[:, None],
                    W2.T, b2[:, None], W_out.T, b_out[:, None])
    return out2.reshape(_B)


# speedup vs baseline: 1.0038x; 1.0038x over previous
import functools

import numpy as np
import jax
import jax.numpy as jnp
from jax import lax
from jax.experimental import pallas as pl
from jax.experimental.pallas import tpu as pltpu
from jax.experimental.pallas import tpu_sc as plsc

_B = 16384
_F = 26
_D = 16
_EMB = _F * _D
_NIX = _F * (_F - 1) // 2
_FEAT_VOCAB = 100000
_PACK = 128 // _D

_K_OF = np.zeros((_F, _F), dtype=np.int64)
_k = 0
for _i in range(_F - 1):
    for _j in range(_i + 1, _F):
        _K_OF[_i, _j] = _k
        _k += 1
_PERM = np.array([_K_OF[f, f + o] for o in range(1, _F) for f in range(_F - o)])


@functools.cache
def _make_sc_gather():
    info = plsc.get_sparse_core_info()
    nw = info.num_cores * info.num_subcores
    bpw = _B // nw
    cs = bpw // 2

    mesh = plsc.VectorSubcoreMesh(core_axis_name="c", subcore_axis_name="s")

    @functools.partial(
        pl.kernel,
        out_type=jax.ShapeDtypeStruct((_F, _B, _D), jnp.float32),
        mesh=mesh,
        scratch_types=[
            pltpu.VMEM((cs,), jnp.int32),
            pltpu.VMEM((cs,), jnp.int32),
            pltpu.VMEM((cs,), jnp.int32),
            pltpu.VMEM((cs, 128), jnp.float32),
            pltpu.VMEM((cs, _D), jnp.float32),
            pltpu.SemaphoreType.DMA,
        ],
        compiler_params=pltpu.CompilerParams(use_tc_tiling_on_sc=True),
    )
    def gather_k(idx_hbm, uid_hbm, ftab_hbm, out_hbm, idx_v, row_v, mod_v,
                 rows8_v, blk_v, sem):
        wid = lax.axis_index("s") * info.num_cores + lax.axis_index("c")

        def extract():
            def group_body(g, carry):
                mods = mod_v[pl.ds(g * 16, 16)]
                for l in range(16):
                    m = mods[l]
                    s = g * 16 + l
                    blk_v[s, :] = rows8_v[s, pl.ds(m, _D)]
                return carry
            lax.fori_loop(0, cs // 16, group_body, 0)

        def chunk_body(i, carry):
            f = i // 2
            base = wid * bpw + (i % 2) * cs
            pltpu.sync_copy(idx_hbm.at[f, 0, pl.ds(base, cs)], idx_v)
            packed_off = jnp.where(f > 0, (f - 1) * (_FEAT_VOCAB // _PACK), 0)
            for i16 in range(cs // 16):
                sl = idx_v[pl.ds(i16 * 16, 16)]
                row_v[pl.ds(i16 * 16, 16)] = (sl >> 3) + packed_off
                mod_v[pl.ds(i16 * 16, 16)] = (sl & 7) * _D

            @pl.when(f == 0)
            def _():
                pltpu.async_copy(uid_hbm.at[row_v], rows8_v, sem).wait()

            @pl.when(f > 0)
            def _():
                pltpu.async_copy(ftab_hbm.at[row_v], rows8_v, sem).wait()

            extract()
            pltpu.sync_copy(blk_v, out_hbm.at[f, pl.ds(base, cs), :])
            return carry

        lax.fori_loop(0, 2 * _F, chunk_body, 0)

    return gather_k


_TB = 512


def _tc_body(emb_ref, w0a_ref, w0b_ref, b0_ref, w1_ref, b1_ref, w2_ref,
             b2_ref, wo_ref, bo_ref, out_ref):
    v = emb_ref[...]
    et = jnp.transpose(v, (0, 2, 1)).reshape(_EMB, _TB)

    slabs = []
    for o in range(1, _F):
        prod = et[: _EMB - _D * o, :] * et[_D * o :, :]
        slabs.append(jnp.sum(prod.reshape(_F - o, _D, _TB), axis=1))
    cross = jnp.concatenate(slabs, axis=0)

    h = jnp.dot(w0a_ref[...], et, preferred_element_type=jnp.float32)
    h = h + jnp.dot(w0b_ref[...], cross, preferred_element_type=jnp.float32)
    h = jnp.maximum(h + b0_ref[...], 0.0)
    h = jnp.maximum(jnp.dot(w1_ref[...], h, preferred_element_type=jnp.float32)
                    + b1_ref[...], 0.0)
    h = jnp.maximum(jnp.dot(w2_ref[...], h, preferred_element_type=jnp.float32)
                    + b2_ref[...], 0.0)
    o_ = jnp.dot(wo_ref[...], h, preferred_element_type=jnp.float32) + bo_ref[...]
    out_ref[...] = jax.nn.sigmoid(o_)[None]


def _tc_call(emb_t, w0a_t, w0b_t, b0c, w1t, b1c, w2t, b2c, wot, boc):
    nt = _B // _TB
    full = lambda shape: pl.BlockSpec(shape, lambda i: (0, 0))
    return pl.pallas_call(
        _tc_body,
        grid=(nt,),
        in_specs=[
            pl.BlockSpec((_F, _TB, _D), lambda i: (0, i, 0)),
            full((400, _EMB)),
            full((400, _NIX)),
            full((400, 1)),
            full((400, 400)),
            full((400, 1)),
            full((400, 400)),
            full((400, 1)),
            full((1, 400)),
            full((1, 1)),
        ],
        out_specs=pl.BlockSpec((1, 1, _TB), lambda i: (i, 0, 0)),
        out_shape=jax.ShapeDtypeStruct((nt, 1, _TB), jnp.float32),
    )(emb_t, w0a_t, w0b_t, b0c, w1t, b1c, w2t, b2c, wot, boc)


def kernel(user_id, feat_0, feat_1, feat_2, feat_3, feat_4, feat_5, feat_6,
           feat_7, feat_8, feat_9, feat_10, feat_11, feat_12, feat_13,
           feat_14, feat_15, feat_16, feat_17, feat_18, feat_19, feat_20,
           feat_21, feat_22, feat_23, feat_24, uid_table, feat_tables,
           W0, b0, W1, b1, W2, b2, W_out, b_out):
    feats = [feat_0, feat_1, feat_2, feat_3, feat_4, feat_5, feat_6, feat_7,
             feat_8, feat_9, feat_10, feat_11, feat_12, feat_13, feat_14,
             feat_15, feat_16, feat_17, feat_18, feat_19, feat_20, feat_21,
             feat_22, feat_23, feat_24]
    idx_all = jnp.stack([user_id] + feats, axis=0)[:, None, :]
    uid8 = uid_table.reshape(-1, 128)
    ftab8 = feat_tables.reshape(-1, 128)

    emb_t = _make_sc_gather()(idx_all, uid8, ftab8)

    w0a_t = W0[:_EMB].T
    w0b_t = W0[_EMB:][_PERM].T
    out2 = _tc_call(emb_t, w0a_t, w0b_t, b0[:, None], W1.T, b1[:, None],
                    W2.T, b2[:, None], W_out.T, b_out[:, None])
    return out2.reshape(_B)


# speedup vs baseline: 1.0503x; 1.0463x over previous
import functools

import numpy as np
import jax
import jax.numpy as jnp
from jax import lax
from jax.experimental import pallas as pl
from jax.experimental.pallas import tpu as pltpu
from jax.experimental.pallas import tpu_sc as plsc

_B = 16384
_F = 26
_D = 16
_EMB = _F * _D
_NIX = _F * (_F - 1) // 2
_FEAT_VOCAB = 100000
_PACK = 128 // _D

_K_OF = np.zeros((_F, _F), dtype=np.int64)
_k = 0
for _i in range(_F - 1):
    for _j in range(_i + 1, _F):
        _K_OF[_i, _j] = _k
        _k += 1
_PERM = np.array([_K_OF[f, f + o] for o in range(1, _F) for f in range(_F - o)])


@functools.cache
def _make_sc_gather():
    info = plsc.get_sparse_core_info()
    nw = info.num_cores * info.num_subcores
    bpw = _B // nw

    mesh = plsc.VectorSubcoreMesh(core_axis_name="c", subcore_axis_name="s")

    @functools.partial(
        pl.kernel,
        out_type=jax.ShapeDtypeStruct((_F, _B, _D), jnp.float32),
        mesh=mesh,
        scratch_types=[
            pltpu.VMEM((bpw,), jnp.int32),
            pltpu.VMEM((bpw, _D), jnp.float32),
            pltpu.SemaphoreType.DMA,
        ],
        compiler_params=pltpu.CompilerParams(use_tc_tiling_on_sc=False),
    )
    def gather_k(idx_hbm, uid_hbm, ftab_hbm, out_hbm, idx_v, rows_v, sem):
        wid = lax.axis_index("s") * info.num_cores + lax.axis_index("c")
        base = wid * bpw

        def field_body(f, carry):
            pltpu.sync_copy(idx_hbm.at[f, 0, pl.ds(base, bpw)], idx_v)

            @pl.when(f == 0)
            def _():
                pltpu.async_copy(uid_hbm.at[idx_v], rows_v, sem).wait()

            @pl.when(f > 0)
            def _():
                pltpu.async_copy(
                    ftab_hbm.at[f - 1].at[idx_v], rows_v, sem).wait()

            pltpu.sync_copy(rows_v, out_hbm.at[f, pl.ds(base, bpw), :])
            return carry

        lax.fori_loop(0, _F, field_body, 0)

    return gather_k


_TB = 512


def _tc_body(emb_ref, w0a_ref, w0b_ref, b0_ref, w1_ref, b1_ref, w2_ref,
             b2_ref, wo_ref, bo_ref, out_ref):
    v = emb_ref[...]
    et = jnp.transpose(v, (0, 2, 1)).reshape(_EMB, _TB)

    slabs = []
    for o in range(1, _F):
        prod = et[: _EMB - _D * o, :] * et[_D * o :, :]
        slabs.append(jnp.sum(prod.reshape(_F - o, _D, _TB), axis=1))
    cross = jnp.concatenate(slabs, axis=0)

    h = jnp.dot(w0a_ref[...], et, preferred_element_type=jnp.float32)
    h = h + jnp.dot(w0b_ref[...], cross, preferred_element_type=jnp.float32)
    h = jnp.maximum(h + b0_ref[...], 0.0)
    h = jnp.maximum(jnp.dot(w1_ref[...], h, preferred_element_type=jnp.float32)
                    + b1_ref[...], 0.0)
    h = jnp.maximum(jnp.dot(w2_ref[...], h, preferred_element_type=jnp.float32)
                    + b2_ref[...], 0.0)
    o_ = jnp.dot(wo_ref[...], h, preferred_element_type=jnp.float32) + bo_ref[...]
    out_ref[...] = jax.nn.sigmoid(o_)[None]


def _tc_call(emb_t, w0a_t, w0b_t, b0c, w1t, b1c, w2t, b2c, wot, boc):
    nt = _B // _TB
    full = lambda shape: pl.BlockSpec(shape, lambda i: (0, 0))
    return pl.pallas_call(
        _tc_body,
        grid=(nt,),
        in_specs=[
            pl.BlockSpec((_F, _TB, _D), lambda i: (0, i, 0)),
            full((400, _EMB)),
            full((400, _NIX)),
            full((400, 1)),
            full((400, 400)),
            full((400, 1)),
            full((400, 400)),
            full((400, 1)),
            full((1, 400)),
            full((1, 1)),
        ],
        out_specs=pl.BlockSpec((1, 1, _TB), lambda i: (i, 0, 0)),
        out_shape=jax.ShapeDtypeStruct((nt, 1, _TB), jnp.float32),
    )(emb_t, w0a_t, w0b_t, b0c, w1t, b1c, w2t, b2c, wot, boc)


def kernel(user_id, feat_0, feat_1, feat_2, feat_3, feat_4, feat_5, feat_6,
           feat_7, feat_8, feat_9, feat_10, feat_11, feat_12, feat_13,
           feat_14, feat_15, feat_16, feat_17, feat_18, feat_19, feat_20,
           feat_21, feat_22, feat_23, feat_24, uid_table, feat_tables,
           W0, b0, W1, b1, W2, b2, W_out, b_out):
    feats = [feat_0, feat_1, feat_2, feat_3, feat_4, feat_5, feat_6, feat_7,
             feat_8, feat_9, feat_10, feat_11, feat_12, feat_13, feat_14,
             feat_15, feat_16, feat_17, feat_18, feat_19, feat_20, feat_21,
             feat_22, feat_23, feat_24]
    idx_all = jnp.stack([user_id] + feats, axis=0)[:, None, :]

    emb_t = _make_sc_gather()(idx_all, uid_table, feat_tables)

    w0a_t = W0[:_EMB].T
    w0b_t = W0[_EMB:][_PERM].T
    out2 = _tc_call(emb_t, w0a_t, w0b_t, b0[:, None], W1.T, b1[:, None],
                    W2.T, b2[:, None], W_out.T, b_out[:, None])
    return out2.reshape(_B)


# speedup vs baseline: 1.1129x; 1.0596x over previous
import functools

import numpy as np
import jax
import jax.numpy as jnp
from jax import lax
from jax.experimental import pallas as pl
from jax.experimental.pallas import tpu as pltpu
from jax.experimental.pallas import tpu_sc as plsc

_B = 16384
_F = 26
_D = 16
_EMB = _F * _D
_NIX = _F * (_F - 1) // 2
_UID_VOCAB = 1000000
_FEAT_VOCAB = 100000
_PACK = 128 // _D

_K_OF = np.zeros((_F, _F), dtype=np.int64)
_k = 0
for _i in range(_F - 1):
    for _j in range(_i + 1, _F):
        _K_OF[_i, _j] = _k
        _k += 1
_PERM = np.array([_K_OF[f, f + o] for o in range(1, _F) for f in range(_F - o)])


def _pack_rows(x):
    c = x.shape[1]
    eye = jnp.eye(_D, dtype=jnp.float32)
    t = jax.lax.dot_general(x, eye, (((0,), (0,)), ((), ())),
                            preferred_element_type=jnp.float32)
    t3 = t.reshape(c // _PACK, _PACK, _D)
    return jnp.concatenate([t3[:, v, :] for v in range(_PACK)], axis=1)


def _repack_body(x_ref, y_ref):
    y_ref[...] = _pack_rows(x_ref[...])


def _repack_uid(uid_t):
    c = 32768
    grid = (pl.cdiv(_UID_VOCAB, c),)
    return pl.pallas_call(
        _repack_body,
        grid=grid,
        in_specs=[pl.BlockSpec((_D, c), lambda i: (0, i))],
        out_specs=pl.BlockSpec((c // _PACK, 128), lambda i: (i, 0)),
        out_shape=jax.ShapeDtypeStruct((_UID_VOCAB // _PACK, 128), jnp.float32),
    )(uid_t)


def _repack_ftab_body(x_ref, y_ref):
    y_ref[...] = _pack_rows(x_ref[...][0])[None]


def _repack_ftab(ftab_t3):
    c = 16384
    grid = (_F - 1, pl.cdiv(_FEAT_VOCAB, c))
    return pl.pallas_call(
        _repack_ftab_body,
        grid=grid,
        in_specs=[pl.BlockSpec((1, _D, c), lambda f, i: (f, 0, i))],
        out_specs=pl.BlockSpec((1, c // _PACK, 128), lambda f, i: (f, i, 0)),
        out_shape=jax.ShapeDtypeStruct(
            (_F - 1, _FEAT_VOCAB // _PACK, 128), jnp.float32),
    )(ftab_t3)


@functools.cache
def _make_sc_gather():
    info = plsc.get_sparse_core_info()
    nw = info.num_cores * info.num_subcores
    bpw = _B // nw
    cs = bpw // 2

    mesh = plsc.VectorSubcoreMesh(core_axis_name="c", subcore_axis_name="s")

    @functools.partial(
        pl.kernel,
        out_type=jax.ShapeDtypeStruct((_F, _B, _D), jnp.float32),
        mesh=mesh,
        scratch_types=[
            pltpu.VMEM((cs,), jnp.int32),
            pltpu.VMEM((cs,), jnp.int32),
            pltpu.VMEM((cs,), jnp.int32),
            pltpu.VMEM((cs, 128), jnp.float32),
            pltpu.VMEM((cs, _D), jnp.float32),
            pltpu.SemaphoreType.DMA,
        ],
        compiler_params=pltpu.CompilerParams(use_tc_tiling_on_sc=True),
    )
    def gather_k(idx_hbm, uid_hbm, ftab_hbm, out_hbm, idx_v, row_v, mod_v,
                 rows8_v, blk_v, sem):
        wid = lax.axis_index("s") * info.num_cores + lax.axis_index("c")

        def extract():
            def group_body(g, carry):
                mods = mod_v[pl.ds(g * 16, 16)]
                for l in range(16):
                    m = mods[l]
                    s = g * 16 + l
                    blk_v[s, :] = rows8_v[s, pl.ds(m, _D)]
                return carry
            lax.fori_loop(0, cs // 16, group_body, 0)

        def chunk_body(i, carry):
            f = i // 2
            base = wid * bpw + (i % 2) * cs
            pltpu.sync_copy(idx_hbm.at[f, 0, pl.ds(base, cs)], idx_v)
            for i16 in range(cs // 16):
                sl = idx_v[pl.ds(i16 * 16, 16)]
                row_v[pl.ds(i16 * 16, 16)] = sl >> 3
                mod_v[pl.ds(i16 * 16, 16)] = (sl & 7) * _D

            @pl.when(f == 0)
            def _():
                pltpu.async_copy(uid_hbm.at[row_v], rows8_v, sem).wait()

            @pl.when(f > 0)
            def _():
                pltpu.async_copy(
                    ftab_hbm.at[f - 1].at[row_v], rows8_v, sem).wait()

            extract()
            pltpu.sync_copy(blk_v, out_hbm.at[f, pl.ds(base, cs), :])
            return carry

        lax.fori_loop(0, 2 * _F, chunk_body, 0)

    return gather_k


_TB = 512


def _tc_body(emb_ref, w0a_ref, w0b_ref, b0_ref, w1_ref, b1_ref, w2_ref,
             b2_ref, wo_ref, bo_ref, out_ref):
    v = emb_ref[...]
    et = jnp.transpose(v, (0, 2, 1)).reshape(_EMB, _TB)

    slabs = []
    for o in range(1, _F):
        prod = et[: _EMB - _D * o, :] * et[_D * o :, :]
        slabs.append(jnp.sum(prod.reshape(_F - o, _D, _TB), axis=1))
    cross = jnp.concatenate(slabs, axis=0)

    h = jnp.dot(w0a_ref[...], et, preferred_element_type=jnp.float32)
    h = h + jnp.dot(w0b_ref[...], cross, preferred_element_type=jnp.float32)
    h = jnp.maximum(h + b0_ref[...], 0.0)
    h = jnp.maximum(jnp.dot(w1_ref[...], h, preferred_element_type=jnp.float32)
                    + b1_ref[...], 0.0)
    h = jnp.maximum(jnp.dot(w2_ref[...], h, preferred_element_type=jnp.float32)
                    + b2_ref[...], 0.0)
    o_ = jnp.dot(wo_ref[...], h, preferred_element_type=jnp.float32) + bo_ref[...]
    out_ref[...] = jax.nn.sigmoid(o_)[None]


def _tc_call(emb, w0a_t, w0b_t, b0c, w1t, b1c, w2t, b2c, wot, boc):
    nt = _B // _TB
    full = lambda shape: pl.BlockSpec(shape, lambda i: (0, 0))
    return pl.pallas_call(
        _tc_body,
        grid=(nt,),
        in_specs=[
            pl.BlockSpec((_F, _TB, _D), lambda i: (0, i, 0)),
            full((400, _EMB)),
            full((400, _NIX)),
            full((400, 1)),
            full((400, 400)),
            full((400, 1)),
            full((400, 400)),
            full((400, 1)),
            full((1, 400)),
            full((1, 1)),
        ],
        out_specs=pl.BlockSpec((1, 1, _TB), lambda i: (i, 0, 0)),
        out_shape=jax.ShapeDtypeStruct((nt, 1, _TB), jnp.float32),
    )(emb, w0a_t, w0b_t, b0c, w1t, b1c, w2t, b2c, wot, boc)


def kernel(user_id, feat_0, feat_1, feat_2, feat_3, feat_4, feat_5, feat_6,
           feat_7, feat_8, feat_9, feat_10, feat_11, feat_12, feat_13,
           feat_14, feat_15, feat_16, feat_17, feat_18, feat_19, feat_20,
           feat_21, feat_22, feat_23, feat_24, uid_table, feat_tables,
           W0, b0, W1, b1, W2, b2, W_out, b_out):
    feats = [feat_0, feat_1, feat_2, feat_3, feat_4, feat_5, feat_6, feat_7,
             feat_8, feat_9, feat_10, feat_11, feat_12, feat_13, feat_14,
             feat_15, feat_16, feat_17, feat_18, feat_19, feat_20, feat_21,
             feat_22, feat_23, feat_24]
    idx_all = jnp.stack([user_id] + feats, axis=0)[:, None, :]

    uid_t = uid_table.T
    ftab_t3 = jnp.transpose(feat_tables, (0, 2, 1))
    uid_pk = _repack_uid(uid_t)
    ftab_pk = _repack_ftab(ftab_t3)

    emb = _make_sc_gather()(idx_all, uid_pk, ftab_pk)

    w0a_t = W0[:_EMB].T
    w0b_t = W0[_EMB:][_PERM].T
    out2 = _tc_call(emb, w0a_t, w0b_t, b0[:, None], W1.T, b1[:, None],
                    W2.T, b2[:, None], W_out.T, b_out[:, None])
    return out2.reshape(_B)


# speedup vs baseline: 1.1413x; 1.0256x over previous
import functools

import numpy as np
import jax
import jax.numpy as jnp
from jax import lax
from jax.experimental import pallas as pl
from jax.experimental.pallas import tpu as pltpu
from jax.experimental.pallas import tpu_sc as plsc

_B = 16384
_F = 26
_D = 16
_EMB = _F * _D
_NIX = _F * (_F - 1) // 2
_UID_VOCAB = 1000000
_FEAT_VOCAB = 100000
_PACK = 128 // _D

_K_OF = np.zeros((_F, _F), dtype=np.int64)
_k = 0
for _i in range(_F - 1):
    for _j in range(_i + 1, _F):
        _K_OF[_i, _j] = _k
        _k += 1
_PERM = np.array([_K_OF[f, f + o] for o in range(1, _F) for f in range(_F - o)])


def _pack_rows(x):
    c = x.shape[1]
    eye = jnp.eye(_D, dtype=jnp.float32)
    t = jnp.dot(x.T, eye, preferred_element_type=jnp.float32)
    t3 = t.reshape(c // _PACK, _PACK, _D)
    return jnp.concatenate([t3[:, v, :] for v in range(_PACK)], axis=1)


def _repack_body(x_ref, y_ref):
    y_ref[...] = _pack_rows(x_ref[...])


def _repack_uid(uid_t):
    c = 32768
    grid = (pl.cdiv(_UID_VOCAB, c),)
    return pl.pallas_call(
        _repack_body,
        grid=grid,
        in_specs=[pl.BlockSpec((_D, c), lambda i: (0, i))],
        out_specs=pl.BlockSpec((c // _PACK, 128), lambda i: (i, 0)),
        out_shape=jax.ShapeDtypeStruct((_UID_VOCAB // _PACK, 128), jnp.float32),
    )(uid_t)


def _repack_ftab_body(x_ref, y_ref):
    y_ref[...] = _pack_rows(x_ref[...][0])[None]


def _repack_ftab(ftab_t3_group):
    nf = ftab_t3_group.shape[0]
    c = 16384
    grid = (nf, pl.cdiv(_FEAT_VOCAB, c))
    return pl.pallas_call(
        _repack_ftab_body,
        grid=grid,
        in_specs=[pl.BlockSpec((1, _D, c), lambda f, i: (f, 0, i))],
        out_specs=pl.BlockSpec((1, c // _PACK, 128), lambda f, i: (f, i, 0)),
        out_shape=jax.ShapeDtypeStruct(
            (nf, _FEAT_VOCAB // _PACK, 128), jnp.float32),
    )(ftab_t3_group)


@functools.cache
def _make_sc_gather(nf, is_uid):
    info = plsc.get_sparse_core_info()
    nw = info.num_cores * info.num_subcores
    bpw = _B // nw
    cs = bpw // 2

    mesh = plsc.VectorSubcoreMesh(core_axis_name="c", subcore_axis_name="s")

    @functools.partial(
        pl.kernel,
        out_type=jax.ShapeDtypeStruct((nf, _B, _D), jnp.float32),
        mesh=mesh,
        scratch_types=[
            pltpu.VMEM((cs,), jnp.int32),
            pltpu.VMEM((cs,), jnp.int32),
            pltpu.VMEM((cs,), jnp.int32),
            pltpu.VMEM((cs, 128), jnp.float32),
            pltpu.VMEM((cs, _D), jnp.float32),
            pltpu.SemaphoreType.DMA,
        ],
        compiler_params=pltpu.CompilerParams(use_tc_tiling_on_sc=True),
    )
    def gather_k(idx_hbm, tab_hbm, out_hbm, idx_v, row_v, mod_v,
                 rows8_v, blk_v, sem):
        wid = lax.axis_index("s") * info.num_cores + lax.axis_index("c")

        def extract():
            def group_body(g, carry):
                mods = mod_v[pl.ds(g * 16, 16)]
                for l in range(16):
                    m = mods[l]
                    s = g * 16 + l
                    blk_v[s, :] = rows8_v[s, pl.ds(m, _D)]
                return carry
            lax.fori_loop(0, cs // 16, group_body, 0)

        def chunk_body(i, carry):
            f = i // 2
            base = wid * bpw + (i % 2) * cs
            pltpu.sync_copy(idx_hbm.at[f, 0, pl.ds(base, cs)], idx_v)
            for i16 in range(cs // 16):
                sl = idx_v[pl.ds(i16 * 16, 16)]
                row_v[pl.ds(i16 * 16, 16)] = sl >> 3
                mod_v[pl.ds(i16 * 16, 16)] = (sl & 7) * _D

            if is_uid:
                pltpu.async_copy(tab_hbm.at[row_v], rows8_v, sem).wait()
            else:
                pltpu.async_copy(
                    tab_hbm.at[f].at[row_v], rows8_v, sem).wait()

            extract()
            pltpu.sync_copy(blk_v, out_hbm.at[f, pl.ds(base, cs), :])
            return carry

        lax.fori_loop(0, 2 * nf, chunk_body, 0)

    return gather_k


_TB = 512


def _tc_body(embu_ref, emba_ref, embb_ref, embc_ref, w0a_ref, w0b_ref,
             b0_ref, w1_ref, b1_ref, w2_ref, b2_ref, wo_ref, bo_ref, out_ref):
    v = jnp.concatenate(
        [embu_ref[...], emba_ref[...], embb_ref[...], embc_ref[...]],
        axis=0)
    et = jnp.transpose(v, (0, 2, 1)).reshape(_EMB, _TB)

    slabs = []
    for o in range(1, _F):
        prod = et[: _EMB - _D * o, :] * et[_D * o :, :]
        slabs.append(jnp.sum(prod.reshape(_F - o, _D, _TB), axis=1))
    cross = jnp.concatenate(slabs, axis=0)

    h = jnp.dot(w0a_ref[...], et, preferred_element_type=jnp.float32)
    h = h + jnp.dot(w0b_ref[...], cross, preferred_element_type=jnp.float32)
    h = jnp.maximum(h + b0_ref[...], 0.0)
    h = jnp.maximum(jnp.dot(w1_ref[...], h, preferred_element_type=jnp.float32)
                    + b1_ref[...], 0.0)
    h = jnp.maximum(jnp.dot(w2_ref[...], h, preferred_element_type=jnp.float32)
                    + b2_ref[...], 0.0)
    o_ = jnp.dot(wo_ref[...], h, preferred_element_type=jnp.float32) + bo_ref[...]
    out_ref[...] = jax.nn.sigmoid(o_)[None]


def _tc_call(embs, w0a_t, w0b_t, b0c, w1t, b1c, w2t, b2c, wot, boc):
    nt = _B // _TB
    full = lambda shape: pl.BlockSpec(shape, lambda i: (0, 0))
    return pl.pallas_call(
        _tc_body,
        grid=(nt,),
        in_specs=[
            pl.BlockSpec((embs[0].shape[0], _TB, _D), lambda i: (0, i, 0)),
            pl.BlockSpec((embs[1].shape[0], _TB, _D), lambda i: (0, i, 0)),
            pl.BlockSpec((embs[2].shape[0], _TB, _D), lambda i: (0, i, 0)),
            pl.BlockSpec((embs[3].shape[0], _TB, _D), lambda i: (0, i, 0)),
            full((400, _EMB)),
            full((400, _NIX)),
            full((400, 1)),
            full((400, 400)),
            full((400, 1)),
            full((400, 400)),
            full((400, 1)),
            full((1, 400)),
            full((1, 1)),
        ],
        out_specs=pl.BlockSpec((1, 1, _TB), lambda i: (i, 0, 0)),
        out_shape=jax.ShapeDtypeStruct((nt, 1, _TB), jnp.float32),
    )(*embs, w0a_t, w0b_t, b0c, w1t, b1c, w2t, b2c, wot, boc)


def kernel(user_id, feat_0, feat_1, feat_2, feat_3, feat_4, feat_5, feat_6,
           feat_7, feat_8, feat_9, feat_10, feat_11, feat_12, feat_13,
           feat_14, feat_15, feat_16, feat_17, feat_18, feat_19, feat_20,
           feat_21, feat_22, feat_23, feat_24, uid_table, feat_tables,
           W0, b0, W1, b1, W2, b2, W_out, b_out):
    feats = [feat_0, feat_1, feat_2, feat_3, feat_4, feat_5, feat_6, feat_7,
             feat_8, feat_9, feat_10, feat_11, feat_12, feat_13, feat_14,
             feat_15, feat_16, feat_17, feat_18, feat_19, feat_20, feat_21,
             feat_22, feat_23, feat_24]
    uid_t = uid_table.T
    ftab_t3 = jnp.transpose(feat_tables, (0, 2, 1))

    idx_u = user_id[None, None, :]
    uid_pk = _repack_uid(uid_t)
    emb_u = _make_sc_gather(1, True)(idx_u, uid_pk)

    groups = [(0, 9), (9, 17), (17, 25)]
    emb_gs = []
    for lo, hi in groups:
        nf = hi - lo
        idx_g = jnp.stack(feats[lo:hi], axis=0)[:, None, :]
        ftab_pk_g = _repack_ftab(ftab_t3[lo:hi])
        emb_gs.append(_make_sc_gather(nf, False)(idx_g, ftab_pk_g))

    embs = [emb_u] + emb_gs

    w0a_t = W0[:_EMB].T
    w0b_t = W0[_EMB:][_PERM].T
    out2 = _tc_call(embs, w0a_t, w0b_t, b0[:, None], W1.T, b1[:, None],
                    W2.T, b2[:, None], W_out.T, b_out[:, None])
    return out2.reshape(_B)


# speedup vs baseline: 1.2025x; 1.0536x over previous
import functools

import numpy as np
import jax
import jax.numpy as jnp
from jax import lax
from jax.experimental import pallas as pl
from jax.experimental.pallas import tpu as pltpu
from jax.experimental.pallas import tpu_sc as plsc

_B = 16384
_F = 26
_D = 16
_EMB = _F * _D
_NIX = _F * (_F - 1) // 2
_UID_VOCAB = 1000000
_FEAT_VOCAB = 100000
_PACK = 128 // _D

_K_OF = np.zeros((_F, _F), dtype=np.int64)
_k = 0
for _i in range(_F - 1):
    for _j in range(_i + 1, _F):
        _K_OF[_i, _j] = _k
        _k += 1
_PERM = np.array([_K_OF[f, f + o] for o in range(1, _F) for f in range(_F - o)])


def _pack_rows(x):
    c = x.shape[1]
    eye = jnp.eye(_D, dtype=jnp.float32)
    t = jnp.dot(x.T, eye, preferred_element_type=jnp.float32)
    t3 = t.reshape(c // _PACK, _PACK, _D)
    return jnp.concatenate([t3[:, v, :] for v in range(_PACK)], axis=1)


def _repack_body(x_ref, y_ref):
    y_ref[...] = _pack_rows(x_ref[...])


def _repack_uid(uid_t):
    c = 32768
    grid = (pl.cdiv(_UID_VOCAB, c),)
    return pl.pallas_call(
        _repack_body,
        grid=grid,
        in_specs=[pl.BlockSpec((_D, c), lambda i: (0, i))],
        out_specs=pl.BlockSpec((c // _PACK, 128), lambda i: (i, 0)),
        out_shape=jax.ShapeDtypeStruct((_UID_VOCAB // _PACK, 128), jnp.float32),
    )(uid_t)


def _repack_ftab_body(x_ref, y_ref):
    y_ref[...] = _pack_rows(x_ref[...][0])[None]


def _repack_ftab(ftab_t3_group):
    nf = ftab_t3_group.shape[0]
    c = 16384
    grid = (nf, pl.cdiv(_FEAT_VOCAB, c))
    return pl.pallas_call(
        _repack_ftab_body,
        grid=grid,
        in_specs=[pl.BlockSpec((1, _D, c), lambda f, i: (f, 0, i))],
        out_specs=pl.BlockSpec((1, c // _PACK, 128), lambda f, i: (f, i, 0)),
        out_shape=jax.ShapeDtypeStruct(
            (nf, _FEAT_VOCAB // _PACK, 128), jnp.float32),
    )(ftab_t3_group)


@functools.cache
def _make_sc_gather(nf, is_uid):
    info = plsc.get_sparse_core_info()
    nw = info.num_cores * info.num_subcores
    bpw = _B // nw
    cs = bpw // 2

    mesh = plsc.VectorSubcoreMesh(core_axis_name="c", subcore_axis_name="s")

    @functools.partial(
        pl.kernel,
        out_type=jax.ShapeDtypeStruct((nf, _B, _D), jnp.float32),
        mesh=mesh,
        scratch_types=[
            pltpu.VMEM((cs,), jnp.int32),
            pltpu.VMEM((cs,), jnp.int32),
            pltpu.VMEM((cs,), jnp.int32),
            pltpu.VMEM((cs, 128), jnp.float32),
            pltpu.VMEM((cs, _D), jnp.float32),
            pltpu.SemaphoreType.DMA,
        ],
        compiler_params=pltpu.CompilerParams(use_tc_tiling_on_sc=True),
    )
    def gather_k(idx_hbm, tab_hbm, out_hbm, idx_v, row_v, mod_v,
                 rows8_v, blk_v, sem):
        wid = lax.axis_index("s") * info.num_cores + lax.axis_index("c")

        def extract():
            def group_body(g, carry):
                mods = mod_v[pl.ds(g * 16, 16)]
                for l in range(16):
                    m = mods[l]
                    s = g * 16 + l
                    blk_v[s, :] = rows8_v[s, pl.ds(m, _D)]
                return carry
            lax.fori_loop(0, cs // 16, group_body, 0)

        def chunk_body(i, carry):
            f = i // 2
            base = wid * bpw + (i % 2) * cs
            pltpu.sync_copy(idx_hbm.at[f, 0, pl.ds(base, cs)], idx_v)
            for i16 in range(cs // 16):
                sl = idx_v[pl.ds(i16 * 16, 16)]
                row_v[pl.ds(i16 * 16, 16)] = sl >> 3
                mod_v[pl.ds(i16 * 16, 16)] = (sl & 7) * _D

            if is_uid:
                pltpu.async_copy(tab_hbm.at[row_v], rows8_v, sem).wait()
            else:
                pltpu.async_copy(
                    tab_hbm.at[f].at[row_v], rows8_v, sem).wait()

            extract()
            pltpu.sync_copy(blk_v, out_hbm.at[f, pl.ds(base, cs), :])
            return carry

        lax.fori_loop(0, 2 * nf, chunk_body, 0)

    return gather_k


_TB = 512


def _tc_body(embu_ref, emba_ref, embb_ref, embc_ref, w0a_ref, w0b_ref,
             b0_ref, w1_ref, b1_ref, w2_ref, b2_ref, wo_ref, bo_ref, out_ref):
    v = jnp.concatenate(
        [embu_ref[...], emba_ref[...], embb_ref[...], embc_ref[...]],
        axis=0)
    et = jnp.transpose(v, (0, 2, 1)).reshape(_EMB, _TB)

    slabs = []
    for o in range(1, _F):
        prod = et[: _EMB - _D * o, :] * et[_D * o :, :]
        slabs.append(jnp.sum(prod.reshape(_F - o, _D, _TB), axis=1))
    cross = jnp.concatenate(slabs, axis=0)

    h = jnp.dot(w0a_ref[...], et, preferred_element_type=jnp.float32)
    h = h + jnp.dot(w0b_ref[...], cross, preferred_element_type=jnp.float32)
    h = jnp.maximum(h + b0_ref[...], 0.0)
    h = jnp.maximum(jnp.dot(w1_ref[...], h, preferred_element_type=jnp.float32)
                    + b1_ref[...], 0.0)
    h = jnp.maximum(jnp.dot(w2_ref[...], h, preferred_element_type=jnp.float32)
                    + b2_ref[...], 0.0)
    o_ = jnp.dot(wo_ref[...], h, preferred_element_type=jnp.float32) + bo_ref[...]
    out_ref[...] = jax.nn.sigmoid(o_)[None]


def _tc_call(embs, w0a_t, w0b_t, b0c, w1t, b1c, w2t, b2c, wot, boc):
    nt = _B // _TB
    full = lambda shape: pl.BlockSpec(shape, lambda i: (0, 0))
    return pl.pallas_call(
        _tc_body,
        grid=(nt,),
        in_specs=[
            pl.BlockSpec((embs[0].shape[0], _TB, _D), lambda i: (0, i, 0)),
            pl.BlockSpec((embs[1].shape[0], _TB, _D), lambda i: (0, i, 0)),
            pl.BlockSpec((embs[2].shape[0], _TB, _D), lambda i: (0, i, 0)),
            pl.BlockSpec((embs[3].shape[0], _TB, _D), lambda i: (0, i, 0)),
            full((400, _EMB)),
            full((400, _NIX)),
            full((400, 1)),
            full((400, 400)),
            full((400, 1)),
            full((400, 400)),
            full((400, 1)),
            full((1, 400)),
            full((1, 1)),
        ],
        out_specs=pl.BlockSpec((1, 1, _TB), lambda i: (i, 0, 0)),
        out_shape=jax.ShapeDtypeStruct((nt, 1, _TB), jnp.float32),
    )(*embs, w0a_t, w0b_t, b0c, w1t, b1c, w2t, b2c, wot, boc)


def kernel(user_id, feat_0, feat_1, feat_2, feat_3, feat_4, feat_5, feat_6,
           feat_7, feat_8, feat_9, feat_10, feat_11, feat_12, feat_13,
           feat_14, feat_15, feat_16, feat_17, feat_18, feat_19, feat_20,
           feat_21, feat_22, feat_23, feat_24, uid_table, feat_tables,
           W0, b0, W1, b1, W2, b2, W_out, b_out):
    feats = [feat_0, feat_1, feat_2, feat_3, feat_4, feat_5, feat_6, feat_7,
             feat_8, feat_9, feat_10, feat_11, feat_12, feat_13, feat_14,
             feat_15, feat_16, feat_17, feat_18, feat_19, feat_20, feat_21,
             feat_22, feat_23, feat_24]
    uid_t = uid_table.T
    ftab_t3 = jnp.transpose(feat_tables, (0, 2, 1))

    idx_u = user_id[None, None, :]
    uid_pk = _repack_uid(uid_t)
    emb_u = _make_sc_gather(1, True)(idx_u, uid_pk)

    groups = [(0, 11), (11, 22), (22, 25)]
    emb_gs = []
    for lo, hi in groups:
        nf = hi - lo
        idx_g = jnp.stack(feats[lo:hi], axis=0)[:, None, :]
        ftab_pk_g = _repack_ftab(ftab_t3[lo:hi])
        emb_gs.append(_make_sc_gather(nf, False)(idx_g, ftab_pk_g))

    embs = [emb_u] + emb_gs

    w0a_t = W0[:_EMB].T
    w0b_t = W0[_EMB:][_PERM].T
    out2 = _tc_call(embs, w0a_t, w0b_t, b0[:, None], W1.T, b1[:, None],
                    W2.T, b2[:, None], W_out.T, b_out[:, None])
    return out2.reshape(_B)
